# baseline (speedup 1.0000x reference)
import functools

import jax
import jax.numpy as jnp
from jax import lax
from jax.experimental import pallas as pl
from jax.experimental.pallas import tpu as pltpu
from jax.experimental.pallas import tpu_sc as plsc

S, D, V, E, F = 2048, 128, 32000, 8, 512
TEMP = 0.5
NS, NV = 4, 25
ST, VT = S // NS, V // NV
FH = F // 2


def _gather_rows(tok_emb, ids):
    info = plsc.get_sparse_core_info()
    nw = info.num_cores * info.num_subcores
    b_per_w = S // nw
    mesh = plsc.VectorSubcoreMesh(core_axis_name="c", subcore_axis_name="s")

    @functools.partial(
        pl.kernel,
        mesh=mesh,
        out_type=jax.ShapeDtypeStruct((S, D), jnp.float32),
        scratch_types=[
            pltpu.VMEM((b_per_w,), jnp.int32),
            pltpu.VMEM((b_per_w, D), jnp.float32),
            pltpu.SemaphoreType.DMA,
        ],
    )
    def k(table_hbm, idx_hbm, out_hbm, idx_v, rows_v, sem):
        wid = lax.axis_index("s") * info.num_cores + lax.axis_index("c")
        base = wid * b_per_w
        pltpu.sync_copy(idx_hbm.at[pl.ds(base, b_per_w)], idx_v)
        pltpu.async_copy(table_hbm.at[idx_v], rows_v, sem).wait()
        pltpu.sync_copy(rows_v, out_hbm.at[pl.ds(base, b_per_w)])

    return k(tok_emb, ids)


def _ln(x, g, b):
    mu = jnp.mean(x, axis=-1, keepdims=True)
    var = jnp.mean((x - mu) ** 2, axis=-1, keepdims=True)
    return (x - mu) * lax.rsqrt(var + 1e-5) * g + b


def _router(h, rw, rb):
    rl = jnp.dot(h, rw, preferred_element_type=jnp.float32)
    rl = (rl + rb) * (1.0 / TEMP)
    m = jnp.max(rl, axis=-1, keepdims=True)
    ex = jnp.exp(rl - m)
    ew = ex / jnp.sum(ex, axis=-1, keepdims=True)

    ids = lax.broadcasted_iota(jnp.int32, (ST, E), 1)
    m1 = jnp.max(ew, axis=-1, keepdims=True)
    i1 = jnp.min(jnp.where(ew == m1, ids, E), axis=-1, keepdims=True)
    ew_mask = jnp.where(ids == i1, -jnp.inf, ew)
    m2 = jnp.max(ew_mask, axis=-1, keepdims=True)
    i2 = jnp.min(jnp.where(ew_mask == m2, ids, E), axis=-1, keepdims=True)
    return ew, jnp.concatenate([i1, i2], axis=-1)


def _half_expert(h_bf, ew, w1_ref, b1_ref, w2_ref, b2_ref, e, half):
    lo = half * FH
    hid = jax.nn.gelu(
        jnp.dot(h_bf, w1_ref[e, :, lo:lo + FH].astype(jnp.bfloat16),
                preferred_element_type=jnp.float32)
        + b1_ref[e, lo:lo + FH])
    eo = jnp.dot(hid.astype(jnp.bfloat16),
                 w2_ref[e, lo:lo + FH, :].astype(jnp.bfloat16),
                 preferred_element_type=jnp.float32)
    if half == 0:
        eo = eo + b2_ref[e]
    return ew[:, e:e + 1] * eo


def _fused_body(te0_ref, te1_ref, pe0_ref, pe1_ref, ing_ref, inb_ref,
                rw_ref, rb_ref, w1_ref, b1_ref, w2_ref, b2_ref,
                outg_ref, outb_ref, pw_ref, pb_ref,
                out_ref, ew_ref, sel_ref,
                h2_scr, h_scr, acc_scr, ew_scr, sel_scr, pw_scr):
    j = pl.program_id(0)
    v = pl.program_id(1)
    cur = j % 2
    nxt_slot = (j + 1) % 2

    @pl.when(jnp.logical_and(j == 0, v == 0))
    def _row0():
        h = _ln(te0_ref[...] + pe0_ref[...], ing_ref[...], inb_ref[...])
        ew, sel = _router(h, rw_ref[...], rb_ref[...])
        ew_scr[0] = ew
        sel_scr[0] = sel
        hb = h.astype(jnp.bfloat16)
        acc = jnp.zeros((ST, D), jnp.float32)
        for e in range(E):
            for half in range(2):
                acc = acc + _half_expert(hb, ew, w1_ref, b1_ref,
                                         w2_ref, b2_ref, e, half)
        h2_scr[0] = _ln(acc, outg_ref[...],
                        outb_ref[...]).astype(jnp.bfloat16)

    nxt = j < NS - 1

    @pl.when(jnp.logical_and(nxt, v == 1))
    def _slice_router():
        h = _ln(te1_ref[...] + pe1_ref[...], ing_ref[...], inb_ref[...])
        h_scr[...] = h
        ew, sel = _router(h, rw_ref[...], rb_ref[...])
        ew_scr[nxt_slot] = ew
        sel_scr[nxt_slot] = sel
        acc_scr[...] = jnp.zeros((ST, D), jnp.float32)

    for k in range(2 * E):
        @pl.when(jnp.logical_and(nxt, v == 2 + k))
        def _slice_expert(k=k):
            hb = h_scr[...].astype(jnp.bfloat16)
            acc_scr[...] = acc_scr[...] + _half_expert(
                hb, ew_scr[nxt_slot], w1_ref, b1_ref, w2_ref, b2_ref,
                k // 2, k % 2)

    @pl.when(jnp.logical_and(nxt, v == 2 * E + 2))
    def _slice_ln():
        h2_scr[nxt_slot] = _ln(acc_scr[...], outg_ref[...],
                               outb_ref[...]).astype(jnp.bfloat16)

    @pl.when(v == 0)
    def _emit_small():
        ew_ref[...] = ew_scr[cur]
        sel_ref[...] = sel_scr[cur]

    @pl.when(j == 0)
    def _proj_row0():
        pwb = pw_ref[...].astype(jnp.bfloat16)
        pw_scr[v] = pwb
        out_ref[...] = (
            jnp.dot(h2_scr[0], pwb, preferred_element_type=jnp.float32)
            + pb_ref[...])

    @pl.when(j > 0)
    def _proj_later():
        out_ref[...] = (
            jnp.dot(h2_scr[cur], pw_scr[v],
                    preferred_element_type=jnp.float32)
            + pb_ref[...])


def kernel(input_ids, tok_emb, pos_emb, in_g, in_b, router_w, router_b,
           w1, b1, w2, b2, out_g, out_b, proj_w, proj_b):
    ids = input_ids.reshape(S).astype(jnp.int32)
    te = _gather_rows(tok_emb, ids)

    logits, ew, sel = pl.pallas_call(
        _fused_body,
        grid=(NS, NV),
        in_specs=[
            pl.BlockSpec((ST, D), lambda j, v: (j, 0)),
            pl.BlockSpec((ST, D),
                         lambda j, v: (jnp.minimum(j + 1, NS - 1), 0)),
            pl.BlockSpec((ST, D), lambda j, v: (j, 0)),
            pl.BlockSpec((ST, D),
                         lambda j, v: (jnp.minimum(j + 1, NS - 1), 0)),
            pl.BlockSpec((D,), lambda j, v: (0,)),
            pl.BlockSpec((D,), lambda j, v: (0,)),
            pl.BlockSpec((D, E), lambda j, v: (0, 0)),
            pl.BlockSpec((E,), lambda j, v: (0,)),
            pl.BlockSpec((E, D, F), lambda j, v: (0, 0, 0)),
            pl.BlockSpec((E, F), lambda j, v: (0, 0)),
            pl.BlockSpec((E, F, D), lambda j, v: (0, 0, 0)),
            pl.BlockSpec((E, D), lambda j, v: (0, 0)),
            pl.BlockSpec((D,), lambda j, v: (0,)),
            pl.BlockSpec((D,), lambda j, v: (0,)),
            pl.BlockSpec((D, VT),
                         lambda j, v: (0, jnp.where(j == 0, v, NV - 1))),
            pl.BlockSpec((1, VT), lambda j, v: (0, v)),
        ],
        out_specs=(
            pl.BlockSpec((ST, VT), lambda j, v: (j, v)),
            pl.BlockSpec((ST, E), lambda j, v: (j, 0)),
            pl.BlockSpec((ST, 2), lambda j, v: (j, 0)),
        ),
        out_shape=(
            jax.ShapeDtypeStruct((S, V), jnp.float32),
            jax.ShapeDtypeStruct((S, E), jnp.float32),
            jax.ShapeDtypeStruct((S, 2), jnp.int32),
        ),
        scratch_shapes=[
            pltpu.VMEM((2, ST, D), jnp.bfloat16),
            pltpu.VMEM((ST, D), jnp.float32),
            pltpu.VMEM((ST, D), jnp.float32),
            pltpu.VMEM((2, ST, E), jnp.float32),
            pltpu.VMEM((2, ST, 2), jnp.int32),
            pltpu.VMEM((NV, D, VT), jnp.bfloat16),
        ],
        compiler_params=pltpu.CompilerParams(
            dimension_semantics=("arbitrary", "arbitrary")),
    )(te, te, pos_emb, pos_emb, in_g, in_b, router_w, router_b,
      w1, b1, w2, b2, out_g, out_b, proj_w, proj_b.reshape(1, V))

    return logits.reshape(1, S, V), ew.reshape(1, S, E), sel.reshape(1, S, 2)


# speedup vs baseline: 1.3267x; 1.3267x over previous
import functools

import jax
import jax.numpy as jnp
from jax import lax
from jax.experimental import pallas as pl
from jax.experimental.pallas import tpu as pltpu
from jax.experimental.pallas import tpu_sc as plsc

S, D, V, E, F = 2048, 128, 32000, 8, 512
TEMP = 0.5
NS, NV = 2, 10
ST, VT = S // NS, V // NV
FH = F // 2


def _gather_rows(tok_emb, ids):
    info = plsc.get_sparse_core_info()
    nw = info.num_cores * info.num_subcores
    b_per_w = S // nw
    mesh = plsc.VectorSubcoreMesh(core_axis_name="c", subcore_axis_name="s")

    @functools.partial(
        pl.kernel,
        mesh=mesh,
        out_type=jax.ShapeDtypeStruct((S, D), jnp.float32),
        scratch_types=[
            pltpu.VMEM((b_per_w,), jnp.int32),
            pltpu.VMEM((b_per_w, D), jnp.float32),
            pltpu.SemaphoreType.DMA,
        ],
    )
    def k(table_hbm, idx_hbm, out_hbm, idx_v, rows_v, sem):
        wid = lax.axis_index("s") * info.num_cores + lax.axis_index("c")
        base = wid * b_per_w
        pltpu.sync_copy(idx_hbm.at[pl.ds(base, b_per_w)], idx_v)
        pltpu.async_copy(table_hbm.at[idx_v], rows_v, sem).wait()
        pltpu.sync_copy(rows_v, out_hbm.at[pl.ds(base, b_per_w)])

    return k(tok_emb, ids)


def _ln(x, g, b):
    mu = jnp.mean(x, axis=-1, keepdims=True)
    var = jnp.mean((x - mu) ** 2, axis=-1, keepdims=True)
    return (x - mu) * lax.rsqrt(var + 1e-5) * g + b


def _router(h, rw, rb):
    rl = jnp.dot(h, rw, preferred_element_type=jnp.float32)
    rl = (rl + rb) * (1.0 / TEMP)
    m = jnp.max(rl, axis=-1, keepdims=True)
    ex = jnp.exp(rl - m)
    ew = ex / jnp.sum(ex, axis=-1, keepdims=True)

    ids = lax.broadcasted_iota(jnp.int32, (ST, E), 1)
    m1 = jnp.max(ew, axis=-1, keepdims=True)
    i1 = jnp.min(jnp.where(ew == m1, ids, E), axis=-1, keepdims=True)
    ew_mask = jnp.where(ids == i1, -jnp.inf, ew)
    m2 = jnp.max(ew_mask, axis=-1, keepdims=True)
    i2 = jnp.min(jnp.where(ew_mask == m2, ids, E), axis=-1, keepdims=True)
    return ew, jnp.concatenate([i1, i2], axis=-1)


def _half_expert(h_bf, ew, w1_ref, b1_ref, w2_ref, b2_ref, e, half):
    lo = half * FH
    hid = jax.nn.gelu(
        jnp.dot(h_bf, w1_ref[e, :, lo:lo + FH].astype(jnp.bfloat16),
                preferred_element_type=jnp.float32)
        + b1_ref[e, lo:lo + FH])
    eo = jnp.dot(hid.astype(jnp.bfloat16),
                 w2_ref[e, lo:lo + FH, :].astype(jnp.bfloat16),
                 preferred_element_type=jnp.float32)
    if half == 0:
        eo = eo + b2_ref[e]
    return ew[:, e:e + 1] * eo


def _fused_body(te0_ref, te1_ref, pe0_ref, pe1_ref, ing_ref, inb_ref,
                rw_ref, rb_ref, w1_ref, b1_ref, w2_ref, b2_ref,
                outg_ref, outb_ref, pw_ref, pb_ref,
                out_ref, ew_ref, sel_ref,
                h2_scr, h_scr, acc_scr, ew_scr, sel_scr, pw_scr):
    j = pl.program_id(0)
    v = pl.program_id(1)
    cur = j % 2
    nxt_slot = (j + 1) % 2

    @pl.when(jnp.logical_and(j == 0, v == 0))
    def _row0():
        h = _ln(te0_ref[...] + pe0_ref[...], ing_ref[...], inb_ref[...])
        ew, sel = _router(h, rw_ref[...], rb_ref[...])
        ew_scr[0] = ew
        sel_scr[0] = sel
        hb = h.astype(jnp.bfloat16)
        acc = jnp.zeros((ST, D), jnp.float32)
        for e in range(E):
            for half in range(2):
                acc = acc + _half_expert(hb, ew, w1_ref, b1_ref,
                                         w2_ref, b2_ref, e, half)
        h2_scr[0] = _ln(acc, outg_ref[...],
                        outb_ref[...]).astype(jnp.bfloat16)

    nxt = j < NS - 1

    @pl.when(jnp.logical_and(nxt, v == 0))
    def _slice_router():
        h = _ln(te1_ref[...] + pe1_ref[...], ing_ref[...], inb_ref[...])
        h_scr[...] = h
        ew, sel = _router(h, rw_ref[...], rb_ref[...])
        ew_scr[nxt_slot] = ew
        sel_scr[nxt_slot] = sel
        acc_scr[...] = jnp.zeros((ST, D), jnp.float32)

    for k in range(E):
        @pl.when(jnp.logical_and(nxt, v == 1 + k))
        def _slice_expert(k=k):
            hb = h_scr[...].astype(jnp.bfloat16)
            for half in range(2):
                acc_scr[...] = acc_scr[...] + _half_expert(
                    hb, ew_scr[nxt_slot], w1_ref, b1_ref, w2_ref, b2_ref,
                    k, half)

    @pl.when(jnp.logical_and(nxt, v == E + 1))
    def _slice_ln():
        h2_scr[nxt_slot] = _ln(acc_scr[...], outg_ref[...],
                               outb_ref[...]).astype(jnp.bfloat16)

    @pl.when(v == 0)
    def _emit_small():
        ew_ref[...] = ew_scr[cur]
        sel_ref[...] = sel_scr[cur]

    @pl.when(j == 0)
    def _proj_row0():
        pwb = pw_ref[...].astype(jnp.bfloat16)
        pw_scr[v] = pwb
        out_ref[...] = (
            jnp.dot(h2_scr[0], pwb, preferred_element_type=jnp.float32)
            + pb_ref[...])

    @pl.when(j > 0)
    def _proj_later():
        out_ref[...] = (
            jnp.dot(h2_scr[cur], pw_scr[v],
                    preferred_element_type=jnp.float32)
            + pb_ref[...])


def kernel(input_ids, tok_emb, pos_emb, in_g, in_b, router_w, router_b,
           w1, b1, w2, b2, out_g, out_b, proj_w, proj_b):
    ids = input_ids.reshape(S).astype(jnp.int32)
    te = _gather_rows(tok_emb, ids)

    logits, ew, sel = pl.pallas_call(
        _fused_body,
        grid=(NS, NV),
        in_specs=[
            pl.BlockSpec((ST, D), lambda j, v: (j, 0)),
            pl.BlockSpec((ST, D),
                         lambda j, v: (jnp.minimum(j + 1, NS - 1), 0)),
            pl.BlockSpec((ST, D), lambda j, v: (j, 0)),
            pl.BlockSpec((ST, D),
                         lambda j, v: (jnp.minimum(j + 1, NS - 1), 0)),
            pl.BlockSpec((D,), lambda j, v: (0,)),
            pl.BlockSpec((D,), lambda j, v: (0,)),
            pl.BlockSpec((D, E), lambda j, v: (0, 0)),
            pl.BlockSpec((E,), lambda j, v: (0,)),
            pl.BlockSpec((E, D, F), lambda j, v: (0, 0, 0)),
            pl.BlockSpec((E, F), lambda j, v: (0, 0)),
            pl.BlockSpec((E, F, D), lambda j, v: (0, 0, 0)),
            pl.BlockSpec((E, D), lambda j, v: (0, 0)),
            pl.BlockSpec((D,), lambda j, v: (0,)),
            pl.BlockSpec((D,), lambda j, v: (0,)),
            pl.BlockSpec((D, VT),
                         lambda j, v: (0, jnp.where(j == 0, v, NV - 1))),
            pl.BlockSpec((1, VT), lambda j, v: (0, v)),
        ],
        out_specs=(
            pl.BlockSpec((ST, VT), lambda j, v: (j, v)),
            pl.BlockSpec((ST, E), lambda j, v: (j, 0)),
            pl.BlockSpec((ST, 2), lambda j, v: (j, 0)),
        ),
        out_shape=(
            jax.ShapeDtypeStruct((S, V), jnp.float32),
            jax.ShapeDtypeStruct((S, E), jnp.float32),
            jax.ShapeDtypeStruct((S, 2), jnp.int32),
        ),
        scratch_shapes=[
            pltpu.VMEM((2, ST, D), jnp.bfloat16),
            pltpu.VMEM((ST, D), jnp.float32),
            pltpu.VMEM((ST, D), jnp.float32),
            pltpu.VMEM((2, ST, E), jnp.float32),
            pltpu.VMEM((2, ST, 2), jnp.int32),
            pltpu.VMEM((NV, D, VT), jnp.bfloat16),
        ],
        compiler_params=pltpu.CompilerParams(
            dimension_semantics=("arbitrary", "arbitrary")),
    )(te, te, pos_emb, pos_emb, in_g, in_b, router_w, router_b,
      w1, b1, w2, b2, out_g, out_b, proj_w, proj_b.reshape(1, V))

    return logits.reshape(1, S, V), ew.reshape(1, S, E), sel.reshape(1, S, 2)
